# baseline (speedup 1.0000x reference)
import functools

import jax
import jax.numpy as jnp
from jax import lax
from jax.experimental import pallas as pl
from jax.experimental.pallas import tpu as pltpu
from jax.experimental.pallas import tpu_sc as plsc

_NC = 2
_NS = 16
_NW = _NC * _NS

_CHUNK = 40
_GRP = 5


def _sc_mesh():
    return plsc.VectorSubcoreMesh(core_axis_name="c", subcore_axis_name="s")


def _make_deg_kernel(NP, E):
    EW = E // _NW
    steps = EW // _CHUNK
    ngrp = steps // _GRP
    RT = NP // _NS

    idx_scratch = [pltpu.VMEM((_CHUNK,), jnp.int32) for _ in range(2 * _GRP)]

    @functools.partial(
        pl.kernel,
        out_type=(jax.ShapeDtypeStruct((_NC * NP,), jnp.float32),
                  jax.ShapeDtypeStruct((_NC * NP,), jnp.float32)),
        mesh=_sc_mesh(),
        scratch_types=idx_scratch + [
            pltpu.VMEM((_CHUNK,), jnp.float32),
            pltpu.VMEM((RT,), jnp.float32),
            pltpu.VMEM_SHARED((NP,), jnp.float32),
            pltpu.VMEM_SHARED((NP,), jnp.float32),
            pltpu.SemaphoreType.DMA,
            pltpu.SemaphoreType.DMA,
        ],
    )
    def deg_kernel(src_hbm, dst_hbm, dout_hbm, din_hbm, *refs):
        idx = refs[:2 * _GRP]
        ones_v, stage_v, dout_sh, din_sh, sem_i, sem_s = refs[2 * _GRP:]
        c = lax.axis_index("c")
        s = lax.axis_index("s")
        wid = s * _NC + c

        offs = sorted({min(o, _CHUNK - 16) for o in range(0, _CHUNK, 16)})
        for o in offs:
            ones_v[pl.ds(o, 16)] = jnp.ones((16,), jnp.float32)

        def zfill(i, _):
            stage_v[pl.ds(i * 16, 16)] = jnp.zeros((16,), jnp.float32)
            return 0
        lax.fori_loop(0, RT // 16, zfill, 0)

        pltpu.sync_copy(stage_v, dout_sh.at[pl.ds(s * RT, RT)])
        pltpu.sync_copy(stage_v, din_sh.at[pl.ds(s * RT, RT)])
        plsc.subcore_barrier()

        def body(g, _):
            lds = []
            for b in range(_GRP):
                base = pl.multiple_of(
                    wid * EW + (g * _GRP + b) * _CHUNK, _CHUNK)
                lds.append(pltpu.async_copy(
                    src_hbm.at[pl.ds(base, _CHUNK)], idx[b], sem_i))
                lds.append(pltpu.async_copy(
                    dst_hbm.at[pl.ds(base, _CHUNK)], idx[_GRP + b], sem_i))
            for d in lds:
                d.wait()
            descs = []
            for b in range(_GRP):
                descs.append(pltpu.async_copy(
                    ones_v, dout_sh.at[idx[b]], sem_s, add=True))
                descs.append(pltpu.async_copy(
                    ones_v, din_sh.at[idx[_GRP + b]], sem_s, add=True))
            for d in descs:
                d.wait()
            return 0
        lax.fori_loop(0, ngrp, body, 0)
        plsc.subcore_barrier()

        pltpu.sync_copy(dout_sh.at[pl.ds(s * RT, RT)], stage_v)
        pltpu.sync_copy(stage_v, dout_hbm.at[pl.ds(c * NP + s * RT, RT)])
        pltpu.sync_copy(din_sh.at[pl.ds(s * RT, RT)], stage_v)
        pltpu.sync_copy(stage_v, din_hbm.at[pl.ds(c * NP + s * RT, RT)])

    return deg_kernel


def _make_segsum_kernel(NP, E, D):
    EW = E // _NW
    steps = EW // _CHUNK
    ngrp = steps // _GRP
    RT = NP // _NS
    ncp = RT // _CHUNK
    assert ngrp >= 4 and ngrp % 2 == 0

    @functools.partial(
        pl.kernel,
        out_type=jax.ShapeDtypeStruct((_NC, NP, D), jnp.float32),
        mesh=_sc_mesh(),
        scratch_types=[
            pltpu.VMEM((_GRP, _CHUNK), jnp.int32),
            pltpu.VMEM((_GRP, _CHUNK), jnp.int32),
            pltpu.VMEM((_GRP, _CHUNK), jnp.int32),
            pltpu.VMEM((_GRP, _CHUNK), jnp.int32),
            pltpu.VMEM((_GRP, _CHUNK, D), jnp.float32),
            pltpu.VMEM_SHARED((NP, D), jnp.float32),
        ] + [pltpu.SemaphoreType.DMA] * (2 + 2 * _GRP),
    )
    def segsum_kernel(h_hbm, src_hbm, dst_hbm, out_hbm,
                      s0, s1, d0, d1, rows_v, acc_sh, *sems):
        sem_i = sems[:2]
        gsem = sems[2:2 + _GRP]
        ssem = sems[2 + _GRP:2 + 2 * _GRP]
        sbanks = (s0, s1)
        dbanks = (d0, d1)
        c = lax.axis_index("c")
        s = lax.axis_index("s")
        wid = s * _NC + c

        pltpu.async_copy(src_hbm.at[wid, 0], s0, sem_i[0])
        pltpu.async_copy(dst_hbm.at[wid, 0], d0, sem_i[0])

        def zfill(i, _):
            for j in range(D // 16):
                rows_v[0, i, pl.ds(j * 16, 16)] = jnp.zeros((16,), jnp.float32)
            return 0
        lax.fori_loop(0, _CHUNK, zfill, 0)

        for k in range(ncp):
            r0 = s * RT + k * _CHUNK
            pltpu.sync_copy(rows_v.at[0], acc_sh.at[pl.ds(r0, _CHUNK)])
        plsc.subcore_barrier()

        def process(g, bank, pf_g, first):
            sb, db = sbanks[bank], dbanks[bank]
            pltpu.make_async_copy(src_hbm.at[wid, g], sb, sem_i[bank]).wait()
            pltpu.make_async_copy(dst_hbm.at[wid, g], db, sem_i[bank]).wait()
            for b in range(_GRP):
                if not first:
                    pltpu.make_async_copy(
                        h_hbm.at[pl.ds(0, _CHUNK)], rows_v.at[b],
                        ssem[b]).wait()
                pltpu.async_copy(h_hbm.at[sb.at[b]], rows_v.at[b], gsem[b])
            if pf_g is not None:
                ob = 1 - bank
                pltpu.async_copy(src_hbm.at[wid, pf_g], sbanks[ob], sem_i[ob])
                pltpu.async_copy(dst_hbm.at[wid, pf_g], dbanks[ob], sem_i[ob])
            for b in range(_GRP):
                pltpu.make_async_copy(
                    h_hbm.at[sb.at[b]], rows_v.at[b], gsem[b]).wait()
                pltpu.async_copy(
                    rows_v.at[b], acc_sh.at[db.at[b]], ssem[b], add=True)

        process(0, 0, 1, True)

        def body(j, _):
            g1 = 2 * j + 1
            process(g1, 1, g1 + 1, False)
            process(g1 + 1, 0, g1 + 2, False)
            return 0
        lax.fori_loop(0, (ngrp - 2) // 2, body, 0)
        process(ngrp - 1, 1, None, False)
        for b in range(_GRP):
            pltpu.make_async_copy(
                h_hbm.at[pl.ds(0, _CHUNK)], rows_v.at[b], ssem[b]).wait()
        plsc.subcore_barrier()

        for k in range(ncp):
            r0 = s * RT + k * _CHUNK
            pltpu.sync_copy(acc_sh.at[pl.ds(r0, _CHUNK)], rows_v.at[0])
            pltpu.sync_copy(rows_v.at[0], out_hbm.at[c, pl.ds(r0, _CHUNK)])

    return segsum_kernel


_ROWS = 1000


def _inv_from_partials(p):
    d = p[0, :, 0] + p[1, :, 0]
    return lax.rsqrt(jnp.maximum(d, 1.0))


def _tc_prep(feat, dout_p):
    N, D = feat.shape
    nb = N // _ROWS

    def body(feat_ref, dp_ref, out_ref):
        inv = _inv_from_partials(dp_ref[...])
        out_ref[...] = feat_ref[...] * inv[:, None]

    return pl.pallas_call(
        body,
        grid=(nb,),
        in_specs=[pl.BlockSpec((_ROWS, D), lambda i: (i, 0)),
                  pl.BlockSpec((2, _ROWS, 1), lambda i: (0, i, 0))],
        out_specs=pl.BlockSpec((_ROWS, D), lambda i: (i, 0)),
        out_shape=jax.ShapeDtypeStruct((N, D), jnp.float32),
    )(feat, dout_p)


def _tc_post(agg_p, din_p, dout_p, bias, gamma, beta, N):
    D = agg_p.shape[2]
    nb = N // _ROWS
    n_f = float(N)

    def body(agg_ref, dinp_ref, doutp_ref, b_ref, g_ref, be_ref,
             out_ref, pre_scr, acc_ref):
        ph = pl.program_id(0)
        i = pl.program_id(1)

        @pl.when(ph == 0)
        def _():
            @pl.when(i == 0)
            def _():
                acc_ref[...] = jnp.zeros_like(acc_ref)

            inv = _inv_from_partials(dinp_ref[...])
            a = agg_ref[...]
            pre = (a[0] + a[1]) * inv[:, None] + b_ref[...][None, :]
            pre_scr[pl.ds(i * _ROWS, _ROWS), :] = pre
            acc_ref[0, :] = acc_ref[0, :] + jnp.sum(pre, axis=0)
            acc_ref[1, :] = acc_ref[1, :] + jnp.sum(pre * pre, axis=0)

        @pl.when(ph == 1)
        def _():
            st = acc_ref[...]
            mu = st[0] / n_f
            var = st[1] / n_f - mu * mu
            scale = lax.rsqrt(var + 1e-5) * g_ref[...]
            shift = be_ref[...] - mu * scale
            inv = _inv_from_partials(doutp_ref[...])
            h = pre_scr[pl.ds(i * _ROWS, _ROWS), :] * scale[None, :] \
                + shift[None, :]
            out_ref[...] = jnp.maximum(h, 0.0) * inv[:, None]

    return pl.pallas_call(
        body,
        grid=(2, nb),
        in_specs=[pl.BlockSpec((2, _ROWS, D), lambda p, i: (0, i, 0)),
                  pl.BlockSpec((2, _ROWS, 1), lambda p, i: (0, i, 0)),
                  pl.BlockSpec((2, _ROWS, 1), lambda p, i: (0, i, 0)),
                  pl.BlockSpec((D,), lambda p, i: (0,)),
                  pl.BlockSpec((D,), lambda p, i: (0,)),
                  pl.BlockSpec((D,), lambda p, i: (0,))],
        out_specs=pl.BlockSpec((_ROWS, D), lambda p, i: (i, 0)),
        out_shape=jax.ShapeDtypeStruct((N, D), jnp.float32),
        scratch_shapes=[pltpu.VMEM((N, D), jnp.float32),
                        pltpu.VMEM((2, D), jnp.float32)],
    )(agg_p, din_p, dout_p, bias, gamma, beta)


def _tc_final(agg_p, din_p, bias, N):
    D = agg_p.shape[2]
    nb = N // _ROWS

    def body(agg_ref, dp_ref, b_ref, out_ref):
        inv = _inv_from_partials(dp_ref[...])
        a = agg_ref[...]
        out_ref[...] = (a[0] + a[1]) * inv[:, None] + b_ref[...][None, :]

    return pl.pallas_call(
        body,
        grid=(nb,),
        in_specs=[pl.BlockSpec((2, _ROWS, D), lambda i: (0, i, 0)),
                  pl.BlockSpec((2, _ROWS, 1), lambda i: (0, i, 0)),
                  pl.BlockSpec((D,), lambda i: (0,))],
        out_specs=pl.BlockSpec((_ROWS, D), lambda i: (i, 0)),
        out_shape=jax.ShapeDtypeStruct((N, D), jnp.float32),
    )(agg_p, din_p, bias)


def kernel(feat, edge_index, b0, b1, b2, g0, beta0, g1, beta1):
    N, D = feat.shape
    E = edge_index.shape[1]
    NP = -(-N // 640) * 640
    EW = E // _NW
    steps = EW // _CHUNK
    assert EW % (_CHUNK * _GRP) == 0 and N % _ROWS == 0 and D % 16 == 0

    ngrp = steps // _GRP
    src4 = edge_index[0].reshape(_NW, ngrp, _GRP, _CHUNK)
    dst4 = edge_index[1].reshape(_NW, ngrp, _GRP, _CHUNK)

    dout_p, din_p = _make_deg_kernel(NP, E)(edge_index[0], edge_index[1])
    dout_p = dout_p.reshape(_NC, NP, 1)
    din_p = din_p.reshape(_NC, NP, 1)
    segsum = _make_segsum_kernel(NP, E, D)

    h = _tc_prep(feat, dout_p)
    for bias, gamma, bshift in ((b0, g0, beta0), (b1, g1, beta1)):
        agg_p = segsum(h, src4, dst4)
        h = _tc_post(agg_p, din_p, dout_p, bias, gamma, bshift, N)
    agg_p = segsum(h, src4, dst4)
    return _tc_final(agg_p, din_p, b2, N)


# speedup vs baseline: 1.0272x; 1.0272x over previous
import functools

import jax
import jax.numpy as jnp
from jax import lax
from jax.experimental import pallas as pl
from jax.experimental.pallas import tpu as pltpu
from jax.experimental.pallas import tpu_sc as plsc

_NC = 2
_NS = 16
_NW = _NC * _NS

_CHUNK = 40
_GRP = 5


def _sc_mesh():
    return plsc.VectorSubcoreMesh(core_axis_name="c", subcore_axis_name="s")


def _make_deg_kernel(NP, E):
    EW = E // _NW
    steps = EW // _CHUNK
    ngrp = steps // _GRP
    RT = NP // _NS
    assert ngrp >= 4 and ngrp % 2 == 0

    idx_scratch = [pltpu.VMEM((_CHUNK,), jnp.int32)
                   for _ in range(2 * 2 * _GRP)]

    @functools.partial(
        pl.kernel,
        out_type=(jax.ShapeDtypeStruct((_NC * NP,), jnp.float32),
                  jax.ShapeDtypeStruct((_NC * NP,), jnp.float32)),
        mesh=_sc_mesh(),
        scratch_types=idx_scratch + [
            pltpu.VMEM((_CHUNK,), jnp.float32),
            pltpu.VMEM((RT,), jnp.float32),
            pltpu.VMEM_SHARED((NP,), jnp.float32),
            pltpu.VMEM_SHARED((NP,), jnp.float32),
            pltpu.SemaphoreType.DMA,
            pltpu.SemaphoreType.DMA,
            pltpu.SemaphoreType.DMA,
            pltpu.SemaphoreType.DMA,
        ],
    )
    def deg_kernel(src_hbm, dst_hbm, dout_hbm, din_hbm, *refs):
        banks = (refs[:2 * _GRP], refs[2 * _GRP:4 * _GRP])
        ones_v, stage_v, dout_sh, din_sh = refs[4 * _GRP:4 * _GRP + 4]
        sem_i = refs[4 * _GRP + 4:4 * _GRP + 6]
        sem_s = refs[4 * _GRP + 6:4 * _GRP + 8]
        c = lax.axis_index("c")
        s = lax.axis_index("s")
        wid = s * _NC + c

        def issue_loads(g, bank):
            idx = banks[bank]
            for b in range(_GRP):
                base = pl.multiple_of(
                    wid * EW + (g * _GRP + b) * _CHUNK, _CHUNK)
                pltpu.async_copy(
                    src_hbm.at[pl.ds(base, _CHUNK)], idx[b], sem_i[bank])
                pltpu.async_copy(
                    dst_hbm.at[pl.ds(base, _CHUNK)], idx[_GRP + b],
                    sem_i[bank])

        def drain_loads(g, bank):
            idx = banks[bank]
            for b in range(_GRP):
                base = pl.multiple_of(
                    wid * EW + (g * _GRP + b) * _CHUNK, _CHUNK)
                pltpu.make_async_copy(
                    src_hbm.at[pl.ds(base, _CHUNK)], idx[b],
                    sem_i[bank]).wait()
                pltpu.make_async_copy(
                    dst_hbm.at[pl.ds(base, _CHUNK)], idx[_GRP + b],
                    sem_i[bank]).wait()

        def drain_scatters(bank):
            idx = banks[bank]
            for b in range(_GRP):
                pltpu.make_async_copy(
                    ones_v, dout_sh.at[idx[b]], sem_s[bank]).wait()
                pltpu.make_async_copy(
                    ones_v, din_sh.at[idx[_GRP + b]], sem_s[bank]).wait()

        issue_loads(0, 0)

        offs = sorted({min(o, _CHUNK - 16) for o in range(0, _CHUNK, 16)})
        for o in offs:
            ones_v[pl.ds(o, 16)] = jnp.ones((16,), jnp.float32)

        def zfill(i, _):
            stage_v[pl.ds(i * 16, 16)] = jnp.zeros((16,), jnp.float32)
            return 0
        lax.fori_loop(0, RT // 16, zfill, 0)

        pltpu.sync_copy(stage_v, dout_sh.at[pl.ds(s * RT, RT)])
        pltpu.sync_copy(stage_v, din_sh.at[pl.ds(s * RT, RT)])
        plsc.subcore_barrier()

        def process(g, bank, pf_g, first):
            idx = banks[bank]
            drain_loads(g, bank)
            for b in range(_GRP):
                pltpu.async_copy(
                    ones_v, dout_sh.at[idx[b]], sem_s[bank], add=True)
                pltpu.async_copy(
                    ones_v, din_sh.at[idx[_GRP + b]], sem_s[bank], add=True)
            if pf_g is not None:
                if not first:
                    drain_scatters(1 - bank)
                issue_loads(pf_g, 1 - bank)

        process(0, 0, 1, True)

        def body(j, _):
            g1 = 2 * j + 1
            process(g1, 1, g1 + 1, False)
            process(g1 + 1, 0, g1 + 2, False)
            return 0
        lax.fori_loop(0, (ngrp - 2) // 2, body, 0)
        process(ngrp - 1, 1, None, False)
        drain_scatters(0)
        drain_scatters(1)
        plsc.subcore_barrier()

        pltpu.sync_copy(dout_sh.at[pl.ds(s * RT, RT)], stage_v)
        pltpu.sync_copy(stage_v, dout_hbm.at[pl.ds(c * NP + s * RT, RT)])
        pltpu.sync_copy(din_sh.at[pl.ds(s * RT, RT)], stage_v)
        pltpu.sync_copy(stage_v, din_hbm.at[pl.ds(c * NP + s * RT, RT)])

    return deg_kernel


def _make_segsum_kernel(NP, E, D):
    EW = E // _NW
    steps = EW // _CHUNK
    ngrp = steps // _GRP
    RT = NP // _NS
    ncp = RT // _CHUNK
    assert ngrp >= 4 and ngrp % 2 == 0

    @functools.partial(
        pl.kernel,
        out_type=jax.ShapeDtypeStruct((_NC, NP, D), jnp.float32),
        mesh=_sc_mesh(),
        scratch_types=[
            pltpu.VMEM((_GRP, _CHUNK), jnp.int32),
            pltpu.VMEM((_GRP, _CHUNK), jnp.int32),
            pltpu.VMEM((_GRP, _CHUNK), jnp.int32),
            pltpu.VMEM((_GRP, _CHUNK), jnp.int32),
            pltpu.VMEM((_GRP, _CHUNK, D), jnp.float32),
            pltpu.VMEM_SHARED((NP, D), jnp.float32),
        ] + [pltpu.SemaphoreType.DMA] * (2 + 2 * _GRP),
    )
    def segsum_kernel(h_hbm, src_hbm, dst_hbm, out_hbm,
                      s0, s1, d0, d1, rows_v, acc_sh, *sems):
        sem_i = sems[:2]
        gsem = sems[2:2 + _GRP]
        ssem = sems[2 + _GRP:2 + 2 * _GRP]
        sbanks = (s0, s1)
        dbanks = (d0, d1)
        c = lax.axis_index("c")
        s = lax.axis_index("s")
        wid = s * _NC + c

        pltpu.async_copy(src_hbm.at[wid, 0], s0, sem_i[0])
        pltpu.async_copy(dst_hbm.at[wid, 0], d0, sem_i[0])

        def zfill(i, _):
            for j in range(D // 16):
                rows_v[0, i, pl.ds(j * 16, 16)] = jnp.zeros((16,), jnp.float32)
            return 0
        lax.fori_loop(0, _CHUNK, zfill, 0)

        for k in range(ncp):
            r0 = s * RT + k * _CHUNK
            pltpu.sync_copy(rows_v.at[0], acc_sh.at[pl.ds(r0, _CHUNK)])
        plsc.subcore_barrier()

        def process(g, bank, pf_g, first):
            sb, db = sbanks[bank], dbanks[bank]
            pltpu.make_async_copy(src_hbm.at[wid, g], sb, sem_i[bank]).wait()
            pltpu.make_async_copy(dst_hbm.at[wid, g], db, sem_i[bank]).wait()
            for b in range(_GRP):
                if not first:
                    pltpu.make_async_copy(
                        h_hbm.at[pl.ds(0, _CHUNK)], rows_v.at[b],
                        ssem[b]).wait()
                pltpu.async_copy(h_hbm.at[sb.at[b]], rows_v.at[b], gsem[b])
            if pf_g is not None:
                ob = 1 - bank
                pltpu.async_copy(src_hbm.at[wid, pf_g], sbanks[ob], sem_i[ob])
                pltpu.async_copy(dst_hbm.at[wid, pf_g], dbanks[ob], sem_i[ob])
            for b in range(_GRP):
                pltpu.make_async_copy(
                    h_hbm.at[sb.at[b]], rows_v.at[b], gsem[b]).wait()
                pltpu.async_copy(
                    rows_v.at[b], acc_sh.at[db.at[b]], ssem[b], add=True)

        process(0, 0, 1, True)

        def body(j, _):
            g1 = 2 * j + 1
            process(g1, 1, g1 + 1, False)
            process(g1 + 1, 0, g1 + 2, False)
            return 0
        lax.fori_loop(0, (ngrp - 2) // 2, body, 0)
        process(ngrp - 1, 1, None, False)
        for b in range(_GRP):
            pltpu.make_async_copy(
                h_hbm.at[pl.ds(0, _CHUNK)], rows_v.at[b], ssem[b]).wait()
        plsc.subcore_barrier()

        for k in range(ncp):
            r0 = s * RT + k * _CHUNK
            pltpu.sync_copy(acc_sh.at[pl.ds(r0, _CHUNK)], rows_v.at[0])
            pltpu.sync_copy(rows_v.at[0], out_hbm.at[c, pl.ds(r0, _CHUNK)])

    return segsum_kernel


_ROWS = 1000


def _inv_from_partials(p):
    d = p[0, :, 0] + p[1, :, 0]
    return lax.rsqrt(jnp.maximum(d, 1.0))


def _tc_prep(feat, dout_p):
    N, D = feat.shape
    nb = N // _ROWS

    def body(feat_ref, dp_ref, out_ref):
        inv = _inv_from_partials(dp_ref[...])
        out_ref[...] = feat_ref[...] * inv[:, None]

    return pl.pallas_call(
        body,
        grid=(nb,),
        in_specs=[pl.BlockSpec((_ROWS, D), lambda i: (i, 0)),
                  pl.BlockSpec((2, _ROWS, 1), lambda i: (0, i, 0))],
        out_specs=pl.BlockSpec((_ROWS, D), lambda i: (i, 0)),
        out_shape=jax.ShapeDtypeStruct((N, D), jnp.float32),
    )(feat, dout_p)


def _tc_post_a(agg_p, din_p, bias, N):
    D = agg_p.shape[2]
    nb = N // _ROWS

    def body(agg_ref, dp_ref, b_ref, pre_ref, stats_ref, acc_ref):
        i = pl.program_id(0)

        @pl.when(i == 0)
        def _():
            acc_ref[...] = jnp.zeros_like(acc_ref)

        inv = _inv_from_partials(dp_ref[...])
        a = agg_ref[...]
        pre = (a[0] + a[1]) * inv[:, None] + b_ref[...][None, :]
        pre_ref[...] = pre
        acc_ref[0, :] = acc_ref[0, :] + jnp.sum(pre, axis=0)
        acc_ref[1, :] = acc_ref[1, :] + jnp.sum(pre * pre, axis=0)
        stats_ref[...] = acc_ref[...]

    return pl.pallas_call(
        body,
        grid=(nb,),
        in_specs=[pl.BlockSpec((2, _ROWS, D), lambda i: (0, i, 0)),
                  pl.BlockSpec((2, _ROWS, 1), lambda i: (0, i, 0)),
                  pl.BlockSpec((D,), lambda i: (0,))],
        out_specs=[pl.BlockSpec((_ROWS, D), lambda i: (i, 0)),
                   pl.BlockSpec((2, D), lambda i: (0, 0))],
        out_shape=[jax.ShapeDtypeStruct((N, D), jnp.float32),
                   jax.ShapeDtypeStruct((2, D), jnp.float32)],
        scratch_shapes=[pltpu.VMEM((2, D), jnp.float32)],
    )(agg_p, din_p, bias)


def _tc_post_b(pre, stats, gamma, beta, dout_p):
    N, D = pre.shape
    nb = N // _ROWS
    n_f = float(N)

    def body(pre_ref, st_ref, g_ref, be_ref, dp_ref, out_ref):
        st = st_ref[...]
        mu = st[0] / n_f
        var = st[1] / n_f - mu * mu
        scale = lax.rsqrt(var + 1e-5) * g_ref[...]
        shift = be_ref[...] - mu * scale
        inv = _inv_from_partials(dp_ref[...])
        h = pre_ref[...] * scale[None, :] + shift[None, :]
        out_ref[...] = jnp.maximum(h, 0.0) * inv[:, None]

    return pl.pallas_call(
        body,
        grid=(nb,),
        in_specs=[pl.BlockSpec((_ROWS, D), lambda i: (i, 0)),
                  pl.BlockSpec((2, D), lambda i: (0, 0)),
                  pl.BlockSpec((D,), lambda i: (0,)),
                  pl.BlockSpec((D,), lambda i: (0,)),
                  pl.BlockSpec((2, _ROWS, 1), lambda i: (0, i, 0))],
        out_specs=pl.BlockSpec((_ROWS, D), lambda i: (i, 0)),
        out_shape=jax.ShapeDtypeStruct((N, D), jnp.float32),
    )(pre, stats, gamma, beta, dout_p)


def _tc_final(agg_p, din_p, bias, N):
    D = agg_p.shape[2]
    nb = N // _ROWS

    def body(agg_ref, dp_ref, b_ref, out_ref):
        inv = _inv_from_partials(dp_ref[...])
        a = agg_ref[...]
        out_ref[...] = (a[0] + a[1]) * inv[:, None] + b_ref[...][None, :]

    return pl.pallas_call(
        body,
        grid=(nb,),
        in_specs=[pl.BlockSpec((2, _ROWS, D), lambda i: (0, i, 0)),
                  pl.BlockSpec((2, _ROWS, 1), lambda i: (0, i, 0)),
                  pl.BlockSpec((D,), lambda i: (0,))],
        out_specs=pl.BlockSpec((_ROWS, D), lambda i: (i, 0)),
        out_shape=jax.ShapeDtypeStruct((N, D), jnp.float32),
    )(agg_p, din_p, bias)


def kernel(feat, edge_index, b0, b1, b2, g0, beta0, g1, beta1):
    N, D = feat.shape
    E = edge_index.shape[1]
    NP = -(-N // 640) * 640
    EW = E // _NW
    steps = EW // _CHUNK
    assert EW % (_CHUNK * _GRP) == 0 and N % _ROWS == 0 and D % 16 == 0

    ngrp = steps // _GRP
    src4 = edge_index[0].reshape(_NW, ngrp, _GRP, _CHUNK)
    dst4 = edge_index[1].reshape(_NW, ngrp, _GRP, _CHUNK)

    dout_p, din_p = _make_deg_kernel(NP, E)(edge_index[0], edge_index[1])
    dout_p = dout_p.reshape(_NC, NP, 1)
    din_p = din_p.reshape(_NC, NP, 1)
    segsum = _make_segsum_kernel(NP, E, D)

    h = _tc_prep(feat, dout_p)
    for bias, gamma, bshift in ((b0, g0, beta0), (b1, g1, beta1)):
        agg_p = segsum(h, src4, dst4)
        pre, stats = _tc_post_a(agg_p, din_p, bias, N)
        h = _tc_post_b(pre, stats, gamma, bshift, dout_p)
    agg_p = segsum(h, src4, dst4)
    return _tc_final(agg_p, din_p, b2, N)


# speedup vs baseline: 1.0533x; 1.0254x over previous
import functools

import jax
import jax.numpy as jnp
from jax import lax
from jax.experimental import pallas as pl
from jax.experimental.pallas import tpu as pltpu
from jax.experimental.pallas import tpu_sc as plsc

_NC = 2
_NS = 16
_NW = _NC * _NS

_CHUNK = 40
_GRP = 5


def _sc_mesh():
    return plsc.VectorSubcoreMesh(core_axis_name="c", subcore_axis_name="s")


def _make_deg_kernel(NP, E):
    EW = E // _NW
    steps = EW // _CHUNK
    ngrp = steps // _GRP
    RT = NP // _NS
    assert ngrp >= 4 and ngrp % 2 == 0

    idx_scratch = [pltpu.VMEM((_CHUNK,), jnp.int32)
                   for _ in range(2 * 2 * _GRP)]

    @functools.partial(
        pl.kernel,
        out_type=(jax.ShapeDtypeStruct((_NC * NP,), jnp.float32),
                  jax.ShapeDtypeStruct((_NC * NP,), jnp.float32)),
        mesh=_sc_mesh(),
        scratch_types=idx_scratch + [
            pltpu.VMEM((_CHUNK,), jnp.float32),
            pltpu.VMEM((RT,), jnp.float32),
            pltpu.VMEM_SHARED((NP,), jnp.float32),
            pltpu.VMEM_SHARED((NP,), jnp.float32),
            pltpu.SemaphoreType.DMA,
            pltpu.SemaphoreType.DMA,
            pltpu.SemaphoreType.DMA,
            pltpu.SemaphoreType.DMA,
        ],
    )
    def deg_kernel(src_hbm, dst_hbm, dout_hbm, din_hbm, *refs):
        banks = (refs[:2 * _GRP], refs[2 * _GRP:4 * _GRP])
        ones_v, stage_v, dout_sh, din_sh = refs[4 * _GRP:4 * _GRP + 4]
        sem_i = refs[4 * _GRP + 4:4 * _GRP + 6]
        sem_s = refs[4 * _GRP + 6:4 * _GRP + 8]
        c = lax.axis_index("c")
        s = lax.axis_index("s")
        wid = s * _NC + c

        def issue_loads(g, bank):
            idx = banks[bank]
            for b in range(_GRP):
                base = pl.multiple_of(
                    wid * EW + (g * _GRP + b) * _CHUNK, _CHUNK)
                pltpu.async_copy(
                    src_hbm.at[pl.ds(base, _CHUNK)], idx[b], sem_i[bank])
                pltpu.async_copy(
                    dst_hbm.at[pl.ds(base, _CHUNK)], idx[_GRP + b],
                    sem_i[bank])

        def drain_loads(g, bank):
            idx = banks[bank]
            for b in range(_GRP):
                base = pl.multiple_of(
                    wid * EW + (g * _GRP + b) * _CHUNK, _CHUNK)
                pltpu.make_async_copy(
                    src_hbm.at[pl.ds(base, _CHUNK)], idx[b],
                    sem_i[bank]).wait()
                pltpu.make_async_copy(
                    dst_hbm.at[pl.ds(base, _CHUNK)], idx[_GRP + b],
                    sem_i[bank]).wait()

        def drain_scatters(bank):
            idx = banks[bank]
            for b in range(_GRP):
                pltpu.make_async_copy(
                    ones_v, dout_sh.at[idx[b]], sem_s[bank]).wait()
                pltpu.make_async_copy(
                    ones_v, din_sh.at[idx[_GRP + b]], sem_s[bank]).wait()

        issue_loads(0, 0)

        offs = sorted({min(o, _CHUNK - 16) for o in range(0, _CHUNK, 16)})
        for o in offs:
            ones_v[pl.ds(o, 16)] = jnp.ones((16,), jnp.float32)

        def zfill(i, _):
            stage_v[pl.ds(i * 16, 16)] = jnp.zeros((16,), jnp.float32)
            return 0
        lax.fori_loop(0, RT // 16, zfill, 0)

        pltpu.sync_copy(stage_v, dout_sh.at[pl.ds(s * RT, RT)])
        pltpu.sync_copy(stage_v, din_sh.at[pl.ds(s * RT, RT)])
        plsc.subcore_barrier()

        def process(g, bank, pf_g, first):
            idx = banks[bank]
            drain_loads(g, bank)
            for b in range(_GRP):
                pltpu.async_copy(
                    ones_v, dout_sh.at[idx[b]], sem_s[bank], add=True)
                pltpu.async_copy(
                    ones_v, din_sh.at[idx[_GRP + b]], sem_s[bank], add=True)
            if pf_g is not None:
                if not first:
                    drain_scatters(1 - bank)
                issue_loads(pf_g, 1 - bank)

        process(0, 0, 1, True)

        def body(j, _):
            g1 = 2 * j + 1
            process(g1, 1, g1 + 1, False)
            process(g1 + 1, 0, g1 + 2, False)
            return 0
        lax.fori_loop(0, (ngrp - 2) // 2, body, 0)
        process(ngrp - 1, 1, None, False)
        drain_scatters(0)
        drain_scatters(1)
        plsc.subcore_barrier()

        pltpu.sync_copy(dout_sh.at[pl.ds(s * RT, RT)], stage_v)
        pltpu.sync_copy(stage_v, dout_hbm.at[pl.ds(c * NP + s * RT, RT)])
        pltpu.sync_copy(din_sh.at[pl.ds(s * RT, RT)], stage_v)
        pltpu.sync_copy(stage_v, din_hbm.at[pl.ds(c * NP + s * RT, RT)])

    return deg_kernel


def _make_segsum_kernel(NP, E, D):
    EW = E // _NW
    steps = EW // _CHUNK
    ngrp = steps // _GRP
    RT = NP // _NS
    ncp = RT // _CHUNK
    assert ngrp >= 4 and ngrp % 2 == 0

    @functools.partial(
        pl.kernel,
        out_type=jax.ShapeDtypeStruct((_NC, NP, D), jnp.float32),
        mesh=_sc_mesh(),
        scratch_types=[
            pltpu.VMEM((_GRP, _CHUNK), jnp.int32),
            pltpu.VMEM((_GRP, _CHUNK), jnp.int32),
            pltpu.VMEM((_GRP, _CHUNK), jnp.int32),
            pltpu.VMEM((_GRP, _CHUNK), jnp.int32),
            pltpu.VMEM((_GRP, _CHUNK, D), jnp.float32),
            pltpu.VMEM_SHARED((NP, D), jnp.float32),
        ] + [pltpu.SemaphoreType.DMA] * (2 + 2 * _GRP),
    )
    def segsum_kernel(h_hbm, src_hbm, dst_hbm, out_hbm,
                      s0, s1, d0, d1, rows_v, acc_sh, *sems):
        sem_i = sems[:2]
        gsem = sems[2:2 + _GRP]
        ssem = sems[2 + _GRP:2 + 2 * _GRP]
        sbanks = (s0, s1)
        dbanks = (d0, d1)
        c = lax.axis_index("c")
        s = lax.axis_index("s")
        wid = s * _NC + c

        pltpu.async_copy(src_hbm.at[wid, 0], s0, sem_i[0])
        pltpu.async_copy(dst_hbm.at[wid, 0], d0, sem_i[0])

        def zfill(i, _):
            for j in range(D // 16):
                rows_v[0, i, pl.ds(j * 16, 16)] = jnp.zeros((16,), jnp.float32)
            return 0
        lax.fori_loop(0, _CHUNK, zfill, 0)

        for k in range(ncp):
            r0 = s * RT + k * _CHUNK
            pltpu.async_copy(rows_v.at[0], acc_sh.at[pl.ds(r0, _CHUNK)],
                             gsem[0])
        for k in range(ncp):
            r0 = s * RT + k * _CHUNK
            pltpu.make_async_copy(rows_v.at[0],
                                  acc_sh.at[pl.ds(r0, _CHUNK)],
                                  gsem[0]).wait()
        plsc.subcore_barrier()

        def process(g, bank, pf_g, first):
            sb, db = sbanks[bank], dbanks[bank]
            pltpu.make_async_copy(src_hbm.at[wid, g], sb, sem_i[bank]).wait()
            pltpu.make_async_copy(dst_hbm.at[wid, g], db, sem_i[bank]).wait()
            for b in range(_GRP):
                if not first:
                    pltpu.make_async_copy(
                        h_hbm.at[pl.ds(0, _CHUNK)], rows_v.at[b],
                        ssem[b]).wait()
                pltpu.async_copy(h_hbm.at[sb.at[b]], rows_v.at[b], gsem[b])
            if pf_g is not None:
                ob = 1 - bank
                pltpu.async_copy(src_hbm.at[wid, pf_g], sbanks[ob], sem_i[ob])
                pltpu.async_copy(dst_hbm.at[wid, pf_g], dbanks[ob], sem_i[ob])
            for b in range(_GRP):
                pltpu.make_async_copy(
                    h_hbm.at[sb.at[b]], rows_v.at[b], gsem[b]).wait()
                pltpu.async_copy(
                    rows_v.at[b], acc_sh.at[db.at[b]], ssem[b], add=True)

        process(0, 0, 1, True)

        def body(j, _):
            g1 = 2 * j + 1
            process(g1, 1, g1 + 1, False)
            process(g1 + 1, 0, g1 + 2, False)
            return 0
        lax.fori_loop(0, (ngrp - 2) // 2, body, 0)
        process(ngrp - 1, 1, None, False)
        for b in range(_GRP):
            pltpu.make_async_copy(
                h_hbm.at[pl.ds(0, _CHUNK)], rows_v.at[b], ssem[b]).wait()
        plsc.subcore_barrier()

        for k in range(ncp):
            b = k % _GRP
            r0 = s * RT + k * _CHUNK
            if k >= _GRP:
                rp = s * RT + (k - _GRP) * _CHUNK
                pltpu.make_async_copy(
                    rows_v.at[b], out_hbm.at[c, pl.ds(rp, _CHUNK)],
                    ssem[b]).wait()
            pltpu.async_copy(acc_sh.at[pl.ds(r0, _CHUNK)], rows_v.at[b],
                             gsem[b])
            pltpu.make_async_copy(acc_sh.at[pl.ds(r0, _CHUNK)],
                                  rows_v.at[b], gsem[b]).wait()
            pltpu.async_copy(rows_v.at[b], out_hbm.at[c, pl.ds(r0, _CHUNK)],
                             ssem[b])
        for k in range(max(ncp - _GRP, 0), ncp):
            b = k % _GRP
            r0 = s * RT + k * _CHUNK
            pltpu.make_async_copy(
                rows_v.at[b], out_hbm.at[c, pl.ds(r0, _CHUNK)],
                ssem[b]).wait()

    return segsum_kernel


_ROWS = 1000


def _inv_from_partials(p):
    d = p[0, :, 0] + p[1, :, 0]
    return lax.rsqrt(jnp.maximum(d, 1.0))


def _tc_prep(feat, dout_p):
    N, D = feat.shape
    nb = N // _ROWS

    def body(feat_ref, dp_ref, out_ref):
        inv = _inv_from_partials(dp_ref[...])
        out_ref[...] = feat_ref[...] * inv[:, None]

    return pl.pallas_call(
        body,
        grid=(nb,),
        in_specs=[pl.BlockSpec((_ROWS, D), lambda i: (i, 0)),
                  pl.BlockSpec((2, _ROWS, 1), lambda i: (0, i, 0))],
        out_specs=pl.BlockSpec((_ROWS, D), lambda i: (i, 0)),
        out_shape=jax.ShapeDtypeStruct((N, D), jnp.float32),
    )(feat, dout_p)


def _tc_post_a(agg_p, din_p, bias, N):
    D = agg_p.shape[2]
    nb = N // _ROWS

    def body(agg_ref, dp_ref, b_ref, pre_ref, stats_ref, acc_ref):
        i = pl.program_id(0)

        @pl.when(i == 0)
        def _():
            acc_ref[...] = jnp.zeros_like(acc_ref)

        inv = _inv_from_partials(dp_ref[...])
        a = agg_ref[...]
        pre = (a[0] + a[1]) * inv[:, None] + b_ref[...][None, :]
        pre_ref[...] = pre
        acc_ref[0, :] = acc_ref[0, :] + jnp.sum(pre, axis=0)
        acc_ref[1, :] = acc_ref[1, :] + jnp.sum(pre * pre, axis=0)
        stats_ref[...] = acc_ref[...]

    return pl.pallas_call(
        body,
        grid=(nb,),
        in_specs=[pl.BlockSpec((2, _ROWS, D), lambda i: (0, i, 0)),
                  pl.BlockSpec((2, _ROWS, 1), lambda i: (0, i, 0)),
                  pl.BlockSpec((D,), lambda i: (0,))],
        out_specs=[pl.BlockSpec((_ROWS, D), lambda i: (i, 0)),
                   pl.BlockSpec((2, D), lambda i: (0, 0))],
        out_shape=[jax.ShapeDtypeStruct((N, D), jnp.float32),
                   jax.ShapeDtypeStruct((2, D), jnp.float32)],
        scratch_shapes=[pltpu.VMEM((2, D), jnp.float32)],
    )(agg_p, din_p, bias)


def _tc_post_b(pre, stats, gamma, beta, dout_p):
    N, D = pre.shape
    nb = N // _ROWS
    n_f = float(N)

    def body(pre_ref, st_ref, g_ref, be_ref, dp_ref, out_ref):
        st = st_ref[...]
        mu = st[0] / n_f
        var = st[1] / n_f - mu * mu
        scale = lax.rsqrt(var + 1e-5) * g_ref[...]
        shift = be_ref[...] - mu * scale
        inv = _inv_from_partials(dp_ref[...])
        h = pre_ref[...] * scale[None, :] + shift[None, :]
        out_ref[...] = jnp.maximum(h, 0.0) * inv[:, None]

    return pl.pallas_call(
        body,
        grid=(nb,),
        in_specs=[pl.BlockSpec((_ROWS, D), lambda i: (i, 0)),
                  pl.BlockSpec((2, D), lambda i: (0, 0)),
                  pl.BlockSpec((D,), lambda i: (0,)),
                  pl.BlockSpec((D,), lambda i: (0,)),
                  pl.BlockSpec((2, _ROWS, 1), lambda i: (0, i, 0))],
        out_specs=pl.BlockSpec((_ROWS, D), lambda i: (i, 0)),
        out_shape=jax.ShapeDtypeStruct((N, D), jnp.float32),
    )(pre, stats, gamma, beta, dout_p)


def _tc_final(agg_p, din_p, bias, N):
    D = agg_p.shape[2]
    nb = N // _ROWS

    def body(agg_ref, dp_ref, b_ref, out_ref):
        inv = _inv_from_partials(dp_ref[...])
        a = agg_ref[...]
        out_ref[...] = (a[0] + a[1]) * inv[:, None] + b_ref[...][None, :]

    return pl.pallas_call(
        body,
        grid=(nb,),
        in_specs=[pl.BlockSpec((2, _ROWS, D), lambda i: (0, i, 0)),
                  pl.BlockSpec((2, _ROWS, 1), lambda i: (0, i, 0)),
                  pl.BlockSpec((D,), lambda i: (0,))],
        out_specs=pl.BlockSpec((_ROWS, D), lambda i: (i, 0)),
        out_shape=jax.ShapeDtypeStruct((N, D), jnp.float32),
    )(agg_p, din_p, bias)


def kernel(feat, edge_index, b0, b1, b2, g0, beta0, g1, beta1):
    N, D = feat.shape
    E = edge_index.shape[1]
    NP = -(-N // 640) * 640
    EW = E // _NW
    steps = EW // _CHUNK
    assert EW % (_CHUNK * _GRP) == 0 and N % _ROWS == 0 and D % 16 == 0

    ngrp = steps // _GRP
    src4 = edge_index[0].reshape(_NW, ngrp, _GRP, _CHUNK)
    dst4 = edge_index[1].reshape(_NW, ngrp, _GRP, _CHUNK)

    dout_p, din_p = _make_deg_kernel(NP, E)(edge_index[0], edge_index[1])
    dout_p = dout_p.reshape(_NC, NP, 1)
    din_p = din_p.reshape(_NC, NP, 1)
    segsum = _make_segsum_kernel(NP, E, D)

    h = _tc_prep(feat, dout_p)
    for bias, gamma, bshift in ((b0, g0, beta0), (b1, g1, beta1)):
        agg_p = segsum(h, src4, dst4)
        pre, stats = _tc_post_a(agg_p, din_p, bias, N)
        h = _tc_post_b(pre, stats, gamma, bshift, dout_p)
    agg_p = segsum(h, src4, dst4)
    return _tc_final(agg_p, din_p, b2, N)
